# baseline (speedup 1.0000x reference)
import functools

import jax
import jax.numpy as jnp
from jax import lax
from jax.experimental import pallas as pl
from jax.experimental.pallas import tpu as pltpu
from jax.experimental.pallas import tpu_sc as plsc

N = 10000
NP = 10112
E = 320000
NC, NS = 2, 16
NW = NC * NS
K = 80
NCH2 = E // NW // K
NCH1 = 2 * NCH2
RPT = NP // NS
ZR = 79

_SC_PARAMS = pltpu.CompilerParams(use_tc_tiling_on_sc=False)


def _fill_rows(ref, rows, d, value):
    offs = list(range(0, d - 15, 16))
    if d % 16:
        offs.append(d - 16)

    def body(r, _):
        for o in offs:
            ref[r, pl.ds(o, 16)] = jnp.full((16,), value, jnp.float32)
        return 0

    lax.fori_loop(0, rows, body, 0)


def _zero_acc(acc, zbuf, d, sid):
    _fill_rows(zbuf, ZR, d, 0.0)
    base = sid * RPT

    def body(i, _):
        pltpu.sync_copy(zbuf, acc.at[pl.ds(base + i * ZR, ZR)])
        return 0

    lax.fori_loop(0, RPT // ZR, body, 0)


_MESH = plsc.VectorSubcoreMesh(core_axis_name="c", subcore_axis_name="s")


@functools.partial(
    pl.kernel,
    out_type=jax.ShapeDtypeStruct((NC, NP, 16), jnp.float32),
    mesh=_MESH,
    compiler_params=_SC_PARAMS,
    scratch_types=[
        pltpu.VMEM((NCH2, K), jnp.int32),
        pltpu.VMEM((K, 16), jnp.float32),
        pltpu.VMEM((ZR, 16), jnp.float32),
        pltpu.VMEM_SHARED((NP, 16), jnp.float32),
    ],
)
def _deg(dst_hbm, out_hbm, dst_v, ones_v, zbuf, acc):
    c = lax.axis_index("c")
    s = lax.axis_index("s")
    wid = s * NC + c
    _zero_acc(acc, zbuf, 16, s)
    _fill_rows(ones_v, K, 16, 1.0)
    pltpu.sync_copy(dst_hbm.at[wid], dst_v)
    plsc.subcore_barrier()

    def body(j, _):
        pltpu.sync_copy(ones_v, acc.at[dst_v.at[j]], add=True)
        return 0

    lax.fori_loop(0, NCH2, body, 0)
    plsc.subcore_barrier()
    pltpu.sync_copy(acc.at[pl.ds(s * RPT, RPT)], out_hbm.at[c, pl.ds(s * RPT, RPT)])


def _make_agg(d, nch, wid_split):
    @functools.partial(
        pl.kernel,
        out_type=jax.ShapeDtypeStruct((NC, NP, d), jnp.float32),
        mesh=_MESH,
        compiler_params=_SC_PARAMS,
        scratch_types=[
            pltpu.VMEM((nch, K), jnp.int32),
            pltpu.VMEM((nch, K), jnp.int32),
            pltpu.VMEM((4, K, d), jnp.float32),
            pltpu.VMEM((ZR, d), jnp.float32),
            pltpu.VMEM_SHARED((NP, d), jnp.float32),
        ] + [pltpu.SemaphoreType.DMA] * 8,
    )
    def agg(g_hbm, src_hbm, dst_hbm, out_hbm, src_v, dst_v, rows_v, zbuf, acc,
            *sems):
        gs, ss = sems[:4], sems[4:]
        c = lax.axis_index("c")
        s = lax.axis_index("s")
        _zero_acc(acc, zbuf, d, s)
        if wid_split:
            widx = s * NC + c
            g = g_hbm
            pltpu.sync_copy(src_hbm.at[widx], src_v)
            pltpu.sync_copy(dst_hbm.at[widx], dst_v)
        else:
            g = g_hbm.at[c]
            pltpu.sync_copy(src_hbm.at[s], src_v.at[pl.ds(0, NCH2)])
            pltpu.sync_copy(src_hbm.at[s + NS], src_v.at[pl.ds(NCH2, NCH2)])
            pltpu.sync_copy(dst_hbm.at[s], dst_v.at[pl.ds(0, NCH2)])
            pltpu.sync_copy(dst_hbm.at[s + NS], dst_v.at[pl.ds(NCH2, NCH2)])
        plsc.subcore_barrier()

        def gather(j, p):
            pltpu.async_copy(g.at[src_v.at[j]], rows_v.at[p], gs[p])

        def gwait(j, p):
            pltpu.make_async_copy(g.at[src_v.at[j]], rows_v.at[p], gs[p]).wait()

        def scat(j, p):
            pltpu.async_copy(rows_v.at[p], acc.at[dst_v.at[j]], ss[p], add=True)

        def swait(j, p):
            pltpu.make_async_copy(rows_v.at[p], acc.at[dst_v.at[j]], ss[p]).wait()

        for p in range(4):
            gather(p, p)

        nmain = (nch // 4) * 4

        def body(jj, _):
            i = 4 * jj
            for p in range(4):
                gwait(i + p, p)
                scat(i + p, p)
            for p in range(4):
                @pl.when(i + p + 4 < nch)
                def _(p=p):
                    swait(i + p, p)
                    gather(i + p + 4, p)
            return 0

        lax.fori_loop(0, nch // 4, body, 0)
        for j in range(nmain, nch):
            gwait(j, j % 4)
            scat(j, j % 4)
        for j in range(nch - 4, nch):
            swait(j, j % 4)
        plsc.subcore_barrier()
        pltpu.sync_copy(acc.at[pl.ds(s * RPT, RPT)],
                        out_hbm.at[c, pl.ds(s * RPT, RPT)])

    return agg


_agg1 = _make_agg(64, NCH1, wid_split=False)
_agg40 = _make_agg(40, NCH2, wid_split=True)


_BMP = NP // 8
_BM = 1000


def _dinv_block(deg_ref):
    deg = deg_ref[...]
    return lax.rsqrt(deg[0, :, 0:1] + deg[1, :, 0:1] + 1.0)


def _row_mask(bm):
    rows = pl.program_id(0) * bm + lax.broadcasted_iota(jnp.int32, (bm, 1), 0)
    return rows < N


def _g1_body(x_ref, w_ref, deg_ref, g_ref):
    dinv = _dinv_block(deg_ref)
    h = jnp.dot(x_ref[...], w_ref[...], preferred_element_type=jnp.float32)
    g = jnp.where(_row_mask(_BMP), h * dinv, 0.0)
    g_ref[0] = g[:, :64]
    g_ref[1] = g[:, 64:]


def _g2_body(agg1_ref, g1_ref, deg_ref, b1_ref, w2_ref, g2_ref):
    dinv = _dinv_block(deg_ref)
    agg = jnp.concatenate([agg1_ref[0], agg1_ref[1]], axis=1)
    g1 = jnp.concatenate([g1_ref[0], g1_ref[1]], axis=1)
    h = jnp.maximum(dinv * (agg + g1) + b1_ref[...], 0.0)
    g2 = jnp.dot(h, w2_ref[...], preferred_element_type=jnp.float32) * dinv
    g2_ref[...] = jnp.where(_row_mask(_BMP), g2, 0.0)


def _out_body(agg2_ref, g2_ref, deg_ref, b2_ref, o_ref):
    dinv = _dinv_block(deg_ref)
    agg = agg2_ref[0] + agg2_ref[1]
    o_ref[...] = dinv * (agg + g2_ref[...]) + b2_ref[...]


def _rows_spec(bm, d):
    return pl.BlockSpec((bm, d), lambda i: (i, 0))


def _pair_spec(bm, d):
    return pl.BlockSpec((2, bm, d), lambda i: (0, i, 0))


def _full_spec(r, d):
    return pl.BlockSpec((r, d), lambda i: (0, 0))


def kernel(x, edge_idx, W1, b1, W2, b2):
    src = edge_idx[0].astype(jnp.int32)
    dst = edge_idx[1].astype(jnp.int32)
    src32 = src.reshape(NW, NCH2, K)
    dst32 = dst.reshape(NW, NCH2, K)

    deg = _deg(dst32)

    g1 = pl.pallas_call(
        _g1_body,
        grid=(8,),
        in_specs=[_rows_spec(_BMP, 128), _full_spec(128, 128),
                  _pair_spec(_BMP, 16)],
        out_specs=_pair_spec(_BMP, 64),
        out_shape=jax.ShapeDtypeStruct((2, NP, 64), jnp.float32),
    )(x, W1, deg)

    agg1 = _agg1(g1, src32, dst32)

    g2 = pl.pallas_call(
        _g2_body,
        grid=(8,),
        in_specs=[_pair_spec(_BMP, 64), _pair_spec(_BMP, 64),
                  _pair_spec(_BMP, 16),
                  _full_spec(1, 128), _full_spec(128, 40)],
        out_specs=_rows_spec(_BMP, 40),
        out_shape=jax.ShapeDtypeStruct((NP, 40), jnp.float32),
    )(agg1, g1, deg, b1.reshape(1, 128), W2)

    agg2 = _agg40(g2, src32, dst32)

    out = pl.pallas_call(
        _out_body,
        grid=(N // _BM,),
        in_specs=[_pair_spec(_BM, 40), _rows_spec(_BM, 40),
                  _pair_spec(_BM, 16), _full_spec(1, 40)],
        out_specs=_rows_spec(_BM, 40),
        out_shape=jax.ShapeDtypeStruct((N, 40), jnp.float32),
    )(agg2, g2, deg, b2.reshape(1, 40))

    return out


# speedup vs baseline: 1.0242x; 1.0242x over previous
import functools

import numpy as np
import jax
import jax.numpy as jnp
from jax import lax
from jax.experimental import pallas as pl
from jax.experimental.pallas import tpu as pltpu
from jax.experimental.pallas import tpu_sc as plsc

N = 10000
NP = 10112
E = 320000
EP = 327680
NC, NS = 2, 16
NW = NC * NS
K = 128
NCH1 = EP // NS // K
NCH2 = EP // NW // K
RPT = NP // NS
ZR = 79

_SC_PARAMS = pltpu.CompilerParams(use_tc_tiling_on_sc=False)


def _fill_rows(ref, rows, d, value):
    offs = list(range(0, d - 15, 16))
    if d % 16:
        offs.append(d - 16)

    def body(r, _):
        for o in offs:
            ref[r, pl.ds(o, 16)] = jnp.full((16,), value, jnp.float32)
        return 0

    lax.fori_loop(0, rows, body, 0)


def _zero_acc(acc, zbuf, d, sid):
    _fill_rows(zbuf, ZR, d, 0.0)
    base = sid * RPT

    def body(i, _):
        pltpu.sync_copy(zbuf, acc.at[pl.ds(base + i * ZR, ZR)])
        return 0

    lax.fori_loop(0, RPT // ZR, body, 0)


_MESH = plsc.VectorSubcoreMesh(core_axis_name="c", subcore_axis_name="s")

def _deg_corr():
    per = EP // NW - E // NW
    fill = (np.arange(per)[None, :] * 89 + np.arange(NW)[:, None] * 997) % N
    counts = np.bincount(fill.ravel(), minlength=NP).astype(np.float32)
    return (1.0 - counts)[:, None]


_DEG_CORR = _deg_corr()


@functools.partial(
    pl.kernel,
    out_type=jax.ShapeDtypeStruct((NC, NP, 16), jnp.float32),
    mesh=_MESH,
    compiler_params=_SC_PARAMS,
    scratch_types=[
        pltpu.VMEM((NCH2, K), jnp.int32),
        pltpu.VMEM((K, 16), jnp.float32),
        pltpu.VMEM((ZR, 16), jnp.float32),
        pltpu.VMEM_SHARED((NP, 16), jnp.float32),
    ],
)
def _deg(dst_hbm, out_hbm, dst_v, ones_v, zbuf, acc):
    c = lax.axis_index("c")
    s = lax.axis_index("s")
    wid = s * NC + c
    _zero_acc(acc, zbuf, 16, s)
    _fill_rows(ones_v, K, 16, 1.0)
    pltpu.sync_copy(dst_hbm.at[wid], dst_v)
    plsc.subcore_barrier()

    def body(j, _):
        pltpu.sync_copy(ones_v, acc.at[dst_v.at[j]], add=True)
        return 0

    lax.fori_loop(0, NCH2, body, 0)
    plsc.subcore_barrier()
    pltpu.sync_copy(acc.at[pl.ds(s * RPT, RPT)], out_hbm.at[c, pl.ds(s * RPT, RPT)])


def _make_agg(d, nch, wid_split):
    @functools.partial(
        pl.kernel,
        out_type=jax.ShapeDtypeStruct((NC, NP, d), jnp.float32),
        mesh=_MESH,
        compiler_params=_SC_PARAMS,
        scratch_types=[
            pltpu.VMEM((nch, K), jnp.int32),
            pltpu.VMEM((nch, K), jnp.int32),
            pltpu.VMEM((4, K, d), jnp.float32),
            pltpu.VMEM((ZR, d), jnp.float32),
            pltpu.VMEM_SHARED((NP, d), jnp.float32),
        ] + [pltpu.SemaphoreType.DMA] * 8,
    )
    def agg(g_hbm, src_hbm, dst_hbm, out_hbm, src_v, dst_v, rows_v, zbuf, acc,
            *sems):
        gs, ss = sems[:4], sems[4:]
        c = lax.axis_index("c")
        s = lax.axis_index("s")
        _zero_acc(acc, zbuf, d, s)
        if wid_split:
            widx = s * NC + c
            g = g_hbm
            pltpu.sync_copy(src_hbm.at[widx], src_v)
            pltpu.sync_copy(dst_hbm.at[widx], dst_v)
        else:
            g = g_hbm.at[c]
            pltpu.sync_copy(src_hbm.at[s], src_v.at[pl.ds(0, NCH2)])
            pltpu.sync_copy(src_hbm.at[s + NS], src_v.at[pl.ds(NCH2, NCH2)])
            pltpu.sync_copy(dst_hbm.at[s], dst_v.at[pl.ds(0, NCH2)])
            pltpu.sync_copy(dst_hbm.at[s + NS], dst_v.at[pl.ds(NCH2, NCH2)])
        plsc.subcore_barrier()

        def gather(j, p):
            pltpu.async_copy(g.at[src_v.at[j]], rows_v.at[p], gs[p])

        def gwait(j, p):
            pltpu.make_async_copy(g.at[src_v.at[j]], rows_v.at[p], gs[p]).wait()

        def scat(j, p):
            pltpu.async_copy(rows_v.at[p], acc.at[dst_v.at[j]], ss[p], add=True)

        def swait(j, p):
            pltpu.make_async_copy(rows_v.at[p], acc.at[dst_v.at[j]], ss[p]).wait()

        for p in range(4):
            gather(p, p)

        def body(jj, _):
            i = 4 * jj
            for p in range(4):
                gwait(i + p, p)
                scat(i + p, p)
            for p in range(4):
                @pl.when(i + p + 4 < nch)
                def _(p=p):
                    swait(i + p, p)
                    gather(i + p + 4, p)
            return 0

        lax.fori_loop(0, nch // 4, body, 0)
        for p in range(4):
            swait(nch - 4 + p, p)
        plsc.subcore_barrier()
        pltpu.sync_copy(acc.at[pl.ds(s * RPT, RPT)],
                        out_hbm.at[c, pl.ds(s * RPT, RPT)])

    return agg


_agg1 = _make_agg(64, NCH1, wid_split=False)
_agg40 = _make_agg(40, NCH2, wid_split=True)


_BMP = NP // 4
_BM = 1000


def _dinv_block(deg_ref, corr_ref):
    deg = deg_ref[...]
    return lax.rsqrt(deg[0, :, 0:1] + deg[1, :, 0:1] + corr_ref[:, 0:1])


def _row_mask(bm):
    rows = pl.program_id(0) * bm + lax.broadcasted_iota(jnp.int32, (bm, 1), 0)
    return rows < N


def _g1_body(x_ref, w_ref, deg_ref, corr_ref, g_ref):
    dinv = _dinv_block(deg_ref, corr_ref)
    h = jnp.dot(x_ref[...], w_ref[...], preferred_element_type=jnp.float32)
    g = jnp.where(_row_mask(_BMP), h * dinv, 0.0)
    g_ref[0] = g[:, :64]
    g_ref[1] = g[:, 64:]


def _g2_body(agg1_ref, g1_ref, deg_ref, corr_ref, b1_ref, w2_ref, g2_ref):
    dinv = _dinv_block(deg_ref, corr_ref)
    agg = jnp.concatenate([agg1_ref[0], agg1_ref[1]], axis=1)
    g1 = jnp.concatenate([g1_ref[0], g1_ref[1]], axis=1)
    h = jnp.maximum(dinv * (agg + g1) + b1_ref[...], 0.0)
    g2 = jnp.dot(h, w2_ref[...], preferred_element_type=jnp.float32) * dinv
    g2_ref[...] = jnp.where(_row_mask(_BMP), g2, 0.0)


def _out_body(agg2_ref, g2_ref, deg_ref, corr_ref, b2_ref, o_ref):
    dinv = _dinv_block(deg_ref, corr_ref)
    agg = agg2_ref[0] + agg2_ref[1]
    o_ref[...] = dinv * (agg + g2_ref[...]) + b2_ref[...]


def _rows_spec(bm, d):
    return pl.BlockSpec((bm, d), lambda i: (i, 0))


def _pair_spec(bm, d):
    return pl.BlockSpec((2, bm, d), lambda i: (0, i, 0))


def _full_spec(r, d):
    return pl.BlockSpec((r, d), lambda i: (0, 0))


def kernel(x, edge_idx, W1, b1, W2, b2):
    src = edge_idx[0].astype(jnp.int32)
    dst = edge_idx[1].astype(jnp.int32)

    def _pad(a, ways, mode):
        per_real, per_tot = E // ways, EP // ways
        per = per_tot - per_real
        if mode == "real":
            fill = (np.arange(per)[None, :] * 89
                    + np.arange(ways)[:, None] * 997) % N
        else:
            fill = N + ((np.arange(per)[None, :]
                         + np.arange(ways)[:, None] * 7) % (NP - N))
        fill = jnp.asarray(fill, jnp.int32)
        return jnp.concatenate([a.reshape(ways, per_real), fill], axis=1)

    src32 = _pad(src, NW, "pad").reshape(NW, NCH2, K)
    dst32 = _pad(dst, NW, "real").reshape(NW, NCH2, K)

    deg = _deg(dst32)
    corr = jnp.asarray(_DEG_CORR)

    g1 = pl.pallas_call(
        _g1_body,
        grid=(4,),
        in_specs=[_rows_spec(_BMP, 128), _full_spec(128, 128),
                  _pair_spec(_BMP, 16), _rows_spec(_BMP, 1)],
        out_specs=_pair_spec(_BMP, 64),
        out_shape=jax.ShapeDtypeStruct((2, NP, 64), jnp.float32),
    )(x, W1, deg, corr)

    agg1 = _agg1(g1, src32, dst32)

    g2 = pl.pallas_call(
        _g2_body,
        grid=(4,),
        in_specs=[_pair_spec(_BMP, 64), _pair_spec(_BMP, 64),
                  _pair_spec(_BMP, 16), _rows_spec(_BMP, 1),
                  _full_spec(1, 128), _full_spec(128, 40)],
        out_specs=_rows_spec(_BMP, 40),
        out_shape=jax.ShapeDtypeStruct((NP, 40), jnp.float32),
    )(agg1, g1, deg, corr, b1.reshape(1, 128), W2)

    agg2 = _agg40(g2, src32, dst32)

    out = pl.pallas_call(
        _out_body,
        grid=(N // _BM,),
        in_specs=[_pair_spec(_BM, 40), _rows_spec(_BM, 40),
                  _pair_spec(_BM, 16), _rows_spec(_BM, 1),
                  _full_spec(1, 40)],
        out_specs=_rows_spec(_BM, 40),
        out_shape=jax.ShapeDtypeStruct((N, 40), jnp.float32),
    )(agg2, g2, deg, corr, b2.reshape(1, 40))

    return out
